# baseline (speedup 1.0000x reference)
import functools

import jax
import jax.numpy as jnp
import numpy as np
from jax import lax
from jax.experimental import pallas as pl
from jax.experimental.pallas import tpu as pltpu
from jax.experimental.pallas import tpu_sc as plsc

L = 2048
H = 16
D = 64
U = 40
RB = 256
NEG = -1e30

_NT = (((1,), (1,)), ((), ()))


def _rotl(x, d):
    return ((x << np.uint32(d)) | (x >> np.uint32(32 - d))).astype(np.uint32)


def _threefry2x32(k1, k2, x0, x1):
    k1 = np.uint32(k1)
    k2 = np.uint32(k2)
    x0 = x0.astype(np.uint32).copy()
    x1 = x1.astype(np.uint32).copy()
    ks = [k1, k2, k1 ^ k2 ^ np.uint32(0x1BD11BDA)]
    rot = [(13, 15, 26, 6), (17, 29, 16, 24)]
    x0 = x0 + ks[0]
    x1 = x1 + ks[1]
    for rs, a, b, c in [(rot[0], 1, 2, 1), (rot[1], 2, 0, 2), (rot[0], 0, 1, 3),
                        (rot[1], 1, 2, 4), (rot[0], 2, 0, 5)]:
        for r in rs:
            x0 = (x0 + x1).astype(np.uint32)
            x1 = x0 ^ _rotl(x1, r)
        x0 = (x0 + ks[a]).astype(np.uint32)
        x1 = (x1 + ks[b] + np.uint32(c)).astype(np.uint32)
    return x0, x1


def _sample_indices() -> np.ndarray:
    b1, b2 = _threefry2x32(0, 42, np.zeros(2, np.uint32), np.arange(2))
    i = np.arange(L * U, dtype=np.uint64)
    o1, o2 = _threefry2x32(b1[1], b2[1],
                           (i >> np.uint64(32)).astype(np.uint32),
                           (i & np.uint64(0xFFFFFFFF)).astype(np.uint32))
    return ((o1 ^ o2).reshape(L, U) % np.uint32(L)).astype(np.int32)


def _build_counts() -> np.ndarray:
    idx = _sample_indices()
    cnt = np.zeros((L, L), np.float32)
    np.add.at(cnt, (np.arange(L)[:, None], idx), 1.0)
    return cnt


_CNT = _build_counts()


def _m_kernel(q_ref, k_ref, cnt_ref, m_ref):
    cnt = cnt_ref[...]
    pos = cnt > 0.0
    for h in range(H):
        q_h = q_ref[:, h * D:(h + 1) * D]
        k_h = k_ref[:, h * D:(h + 1) * D]
        c = jax.lax.dot_general(q_h, k_h, _NT,
                                preferred_element_type=jnp.float32)
        m_max = jnp.max(jnp.where(pos, c, NEG), axis=1)
        m_sum = jnp.sum(c * cnt, axis=1)
        m_ref[h, 0, :] = m_max - m_sum * (1.0 / L)


UPAD = 48
_NLANE = 16


_GDN = lax.GatherDimensionNumbers(offset_dims=(), collapsed_slice_dims=(0,),
                                  start_index_map=(0,))


def _lane_shuffle(x, idx16):
    return lax.gather(x, idx16[:, None], _GDN, (1,),
                      mode=lax.GatherScatterMode.PROMISE_IN_BOUNDS)


def _sc_topk_kernel(m_hbm, out_hbm, m_v, idx_v):
    wid = lax.axis_index("s") * 2 + lax.axis_index("c")

    @pl.when(wid < H)
    def _():
        pltpu.sync_copy(m_hbm.at[wid], m_v)
        zero16 = jnp.zeros((_NLANE,), jnp.int32)
        for j in range(128 // _NLANE):
            idx_v[pl.ds(j * _NLANE, _NLANE)] = zero16
        lane_iota = lax.iota(jnp.int32, _NLANE)

        def pick_one(u, _):
            def scan_chunk(i, carry):
                rmax, ridx = carry
                v = m_v[pl.ds(i * _NLANE, _NLANE)]
                take = v > rmax
                rmax = jnp.where(take, v, rmax)
                ridx = jnp.where(take, i * _NLANE + lane_iota, ridx)
                return rmax, ridx

            rmax0 = jnp.full((_NLANE,), NEG, jnp.float32)
            ridx0 = jnp.zeros((_NLANE,), jnp.int32)
            rmax, ridx = lax.fori_loop(0, L // _NLANE, scan_chunk,
                                       (rmax0, ridx0))
            cur = rmax
            for sh in (1, 2, 4, 8):
                cur = jnp.maximum(cur, _lane_shuffle(cur, lane_iota ^ sh))
            cand = jnp.where(rmax == cur, ridx, L)
            for sh in (1, 2, 4, 8):
                cand = jnp.minimum(cand, _lane_shuffle(cand, lane_iota ^ sh))
            idx = cand[0]

            ub = (u // _NLANE) * _NLANE
            ichunk = idx_v[pl.ds(ub, _NLANE)]
            idx_v[pl.ds(ub, _NLANE)] = jnp.where(lane_iota == u - ub,
                                                 cand, ichunk)
            mb = (idx // _NLANE) * _NLANE
            mchunk = m_v[pl.ds(mb, _NLANE)]
            m_v[pl.ds(mb, _NLANE)] = jnp.where(lane_iota == idx - mb,
                                               jnp.float32(NEG), mchunk)
            return 0

        lax.fori_loop(0, U, pick_one, 0)
        pltpu.sync_copy(idx_v, out_hbm.at[wid])


def _sc_topk(m):
    mesh = plsc.VectorSubcoreMesh(core_axis_name="c", subcore_axis_name="s")
    fn = functools.partial(
        pl.kernel,
        out_type=jax.ShapeDtypeStruct((H, 128), jnp.int32),
        mesh=mesh,
        scratch_types=[
            pltpu.VMEM((L,), jnp.float32),
            pltpu.VMEM((128,), jnp.int32),
        ],
    )(_sc_topk_kernel)
    return fn(m)


def _attn_kernel(idx_ref, q_ref, k_ref, v_ref, out_ref):
    idx_all = idx_ref[...]

    iota_l = jax.lax.broadcasted_iota(jnp.int32, (U, L), 1)
    iota_s = jax.lax.broadcasted_iota(jnp.int32, (U, U), 0)
    iota_u = jax.lax.broadcasted_iota(jnp.int32, (U, U), 1)
    for h in range(H):
        q_h = q_ref[:, h * D:(h + 1) * D]
        k_h = k_ref[:, h * D:(h + 1) * D]
        v_h = v_ref[:, h * D:(h + 1) * D]
        idx_row = idx_all[h:h + 1, :]
        idx_col = jnp.transpose(idx_row, (1, 0))

        g = (iota_l == idx_col).astype(jnp.float32)
        q_red = jnp.dot(g, q_h, preferred_element_type=jnp.float32)

        scores = jax.lax.dot_general(
            q_red, k_h, _NT,
            preferred_element_type=jnp.float32) * (1.0 / np.sqrt(D))
        s_max = jnp.max(scores, axis=1, keepdims=True)
        e = jnp.exp(scores - s_max)
        p = e / jnp.sum(e, axis=1, keepdims=True)
        upd = jnp.dot(p, v_h, preferred_element_type=jnp.float32)

        clip_row = jnp.minimum(idx_row, U - 1)
        eq = clip_row == iota_s
        u_star = jnp.max(jnp.where(eq, iota_u, -1), axis=1, keepdims=True)
        w = ((iota_u == u_star) & eq).astype(jnp.float32)
        scat = jnp.dot(w, upd, preferred_element_type=jnp.float32)

        v_mean = jnp.mean(v_h, axis=0, keepdims=True)
        out_ref[h] = jnp.where(u_star < 0,
                               jnp.broadcast_to(v_mean, (U, D)), scat)


@jax.jit
def _run(queries, keys, values):
    q2 = queries.reshape(L, H * D)
    k2 = keys.reshape(L, H * D)
    v2 = values.reshape(L, H * D)
    cnt = jnp.asarray(_CNT)

    m = pl.pallas_call(
        _m_kernel,
        grid=(L // RB,),
        in_specs=[
            pl.BlockSpec((RB, H * D), lambda rb: (rb, 0)),
            pl.BlockSpec((L, H * D), lambda rb: (0, 0)),
            pl.BlockSpec((RB, L), lambda rb: (rb, 0)),
        ],
        out_specs=pl.BlockSpec((H, 1, RB), lambda rb: (0, 0, rb)),
        out_shape=jax.ShapeDtypeStruct((H, 1, L), jnp.float32),
    )(q2, k2, cnt)

    idx_all = _sc_topk(m.reshape(H, L))[:, :U]

    ctx = pl.pallas_call(
        _attn_kernel,
        grid=(1,),
        in_specs=[
            pl.BlockSpec((H, U), lambda i: (0, 0)),
            pl.BlockSpec((L, H * D), lambda i: (0, 0)),
            pl.BlockSpec((L, H * D), lambda i: (0, 0)),
            pl.BlockSpec((L, H * D), lambda i: (0, 0)),
        ],
        out_specs=pl.BlockSpec((H, U, D), lambda i: (0, 0, 0)),
        out_shape=jax.ShapeDtypeStruct((H, U, D), jnp.float32),
    )(idx_all, q2, k2, v2)

    return jnp.transpose(ctx, (1, 0, 2))[None]


def kernel(queries, keys, values, attn_mask):
    return (_run(queries, keys, values), None)


# speedup vs baseline: 1.1085x; 1.1085x over previous
import functools

import jax
import jax.numpy as jnp
import numpy as np
from jax import lax
from jax.experimental import pallas as pl
from jax.experimental.pallas import tpu as pltpu
from jax.experimental.pallas import tpu_sc as plsc

L = 2048
H = 16
D = 64
U = 40
RB = 256
NEG = -1e30

_NT = (((1,), (1,)), ((), ()))


def _rotl(x, d):
    return ((x << np.uint32(d)) | (x >> np.uint32(32 - d))).astype(np.uint32)


def _threefry2x32(k1, k2, x0, x1):
    k1 = np.uint32(k1)
    k2 = np.uint32(k2)
    x0 = x0.astype(np.uint32).copy()
    x1 = x1.astype(np.uint32).copy()
    ks = [k1, k2, k1 ^ k2 ^ np.uint32(0x1BD11BDA)]
    rot = [(13, 15, 26, 6), (17, 29, 16, 24)]
    x0 = x0 + ks[0]
    x1 = x1 + ks[1]
    for rs, a, b, c in [(rot[0], 1, 2, 1), (rot[1], 2, 0, 2), (rot[0], 0, 1, 3),
                        (rot[1], 1, 2, 4), (rot[0], 2, 0, 5)]:
        for r in rs:
            x0 = (x0 + x1).astype(np.uint32)
            x1 = x0 ^ _rotl(x1, r)
        x0 = (x0 + ks[a]).astype(np.uint32)
        x1 = (x1 + ks[b] + np.uint32(c)).astype(np.uint32)
    return x0, x1


def _sample_indices() -> np.ndarray:
    b1, b2 = _threefry2x32(0, 42, np.zeros(2, np.uint32), np.arange(2))
    i = np.arange(L * U, dtype=np.uint64)
    o1, o2 = _threefry2x32(b1[1], b2[1],
                           (i >> np.uint64(32)).astype(np.uint32),
                           (i & np.uint64(0xFFFFFFFF)).astype(np.uint32))
    return ((o1 ^ o2).reshape(L, U) % np.uint32(L)).astype(np.int32)


def _build_counts() -> np.ndarray:
    idx = _sample_indices()
    cnt = np.zeros((L, L), np.float32)
    np.add.at(cnt, (np.arange(L)[:, None], idx), 1.0)
    return cnt


_CNT = _build_counts()


def _m_kernel(q_ref, k_ref, cnt_ref, m_ref):
    cnt = cnt_ref[...]
    pos = cnt > 0.0
    for h in range(H):
        q_h = q_ref[:, h * D:(h + 1) * D]
        k_h = k_ref[:, h * D:(h + 1) * D]
        c = jax.lax.dot_general(q_h, k_h, _NT,
                                preferred_element_type=jnp.float32)
        m_max = jnp.max(jnp.where(pos, c, NEG), axis=1)
        m_sum = jnp.sum(c * cnt, axis=1)
        m_ref[h, 0, :] = m_max - m_sum * (1.0 / L)


UPAD = 48
_NLANE = 16


_GDN = lax.GatherDimensionNumbers(offset_dims=(), collapsed_slice_dims=(0,),
                                  start_index_map=(0,))


def _lane_shuffle(x, idx16):
    return lax.gather(x, idx16[:, None], _GDN, (1,),
                      mode=lax.GatherScatterMode.PROMISE_IN_BOUNDS)


def _sc_topk_kernel(m_hbm, out_hbm, m_v, idx_v):
    wid = lax.axis_index("s") * 2 + lax.axis_index("c")

    @pl.when(wid < H)
    def _():
        pltpu.sync_copy(m_hbm.at[wid], m_v)
        zero16 = jnp.zeros((_NLANE,), jnp.int32)
        for j in range(128 // _NLANE):
            idx_v[pl.ds(j * _NLANE, _NLANE)] = zero16
        lane_iota = lax.iota(jnp.int32, _NLANE)

        def pick_one(u, _):
            def scan_chunk(i, carry):
                rmax, ridx = carry
                v = m_v[pl.ds(i * _NLANE, _NLANE)]
                take = v > rmax
                rmax = jnp.where(take, v, rmax)
                ridx = jnp.where(take, i * _NLANE + lane_iota, ridx)
                return rmax, ridx

            rmax0 = jnp.full((_NLANE,), NEG, jnp.float32)
            ridx0 = jnp.zeros((_NLANE,), jnp.int32)
            rmax, ridx = lax.fori_loop(0, L // _NLANE, scan_chunk,
                                       (rmax0, ridx0), unroll=8)
            cur = rmax
            for sh in (1, 2, 4, 8):
                cur = jnp.maximum(cur, _lane_shuffle(cur, lane_iota ^ sh))
            cand = jnp.where(rmax == cur, ridx, L)
            for sh in (1, 2, 4, 8):
                cand = jnp.minimum(cand, _lane_shuffle(cand, lane_iota ^ sh))
            idx = cand[0]

            ub = (u // _NLANE) * _NLANE
            ichunk = idx_v[pl.ds(ub, _NLANE)]
            idx_v[pl.ds(ub, _NLANE)] = jnp.where(lane_iota == u - ub,
                                                 cand, ichunk)
            mb = (idx // _NLANE) * _NLANE
            mchunk = m_v[pl.ds(mb, _NLANE)]
            m_v[pl.ds(mb, _NLANE)] = jnp.where(lane_iota == idx - mb,
                                               jnp.float32(NEG), mchunk)
            return 0

        lax.fori_loop(0, U, pick_one, 0)
        pltpu.sync_copy(idx_v, out_hbm.at[wid])


def _sc_topk(m):
    mesh = plsc.VectorSubcoreMesh(core_axis_name="c", subcore_axis_name="s")
    fn = functools.partial(
        pl.kernel,
        out_type=jax.ShapeDtypeStruct((H, 128), jnp.int32),
        mesh=mesh,
        scratch_types=[
            pltpu.VMEM((L,), jnp.float32),
            pltpu.VMEM((128,), jnp.int32),
        ],
    )(_sc_topk_kernel)
    return fn(m)


def _attn_kernel(idx_ref, q_ref, k_ref, v_ref, out_ref):
    idx_all = idx_ref[...]

    iota_l = jax.lax.broadcasted_iota(jnp.int32, (U, L), 1)
    iota_s = jax.lax.broadcasted_iota(jnp.int32, (U, U), 0)
    iota_u = jax.lax.broadcasted_iota(jnp.int32, (U, U), 1)
    for h in range(H):
        q_h = q_ref[:, h * D:(h + 1) * D]
        k_h = k_ref[:, h * D:(h + 1) * D]
        v_h = v_ref[:, h * D:(h + 1) * D]
        idx_row = idx_all[h:h + 1, :]
        idx_col = jnp.transpose(idx_row, (1, 0))

        g = (iota_l == idx_col).astype(jnp.float32)
        q_red = jnp.dot(g, q_h, preferred_element_type=jnp.float32)

        scores = jax.lax.dot_general(
            q_red, k_h, _NT,
            preferred_element_type=jnp.float32) * (1.0 / np.sqrt(D))
        s_max = jnp.max(scores, axis=1, keepdims=True)
        e = jnp.exp(scores - s_max)
        p = e / jnp.sum(e, axis=1, keepdims=True)
        upd = jnp.dot(p, v_h, preferred_element_type=jnp.float32)

        clip_row = jnp.minimum(idx_row, U - 1)
        eq = clip_row == iota_s
        u_star = jnp.max(jnp.where(eq, iota_u, -1), axis=1, keepdims=True)
        w = ((iota_u == u_star) & eq).astype(jnp.float32)
        scat = jnp.dot(w, upd, preferred_element_type=jnp.float32)

        v_mean = jnp.mean(v_h, axis=0, keepdims=True)
        out_ref[h] = jnp.where(u_star < 0,
                               jnp.broadcast_to(v_mean, (U, D)), scat)


@jax.jit
def _run(queries, keys, values):
    q2 = queries.reshape(L, H * D)
    k2 = keys.reshape(L, H * D)
    v2 = values.reshape(L, H * D)
    cnt = jnp.asarray(_CNT)

    m = pl.pallas_call(
        _m_kernel,
        grid=(L // RB,),
        in_specs=[
            pl.BlockSpec((RB, H * D), lambda rb: (rb, 0)),
            pl.BlockSpec((L, H * D), lambda rb: (0, 0)),
            pl.BlockSpec((RB, L), lambda rb: (rb, 0)),
        ],
        out_specs=pl.BlockSpec((H, 1, RB), lambda rb: (0, 0, rb)),
        out_shape=jax.ShapeDtypeStruct((H, 1, L), jnp.float32),
    )(q2, k2, cnt)

    idx_all = _sc_topk(m.reshape(H, L))[:, :U]

    ctx = pl.pallas_call(
        _attn_kernel,
        grid=(1,),
        in_specs=[
            pl.BlockSpec((H, U), lambda i: (0, 0)),
            pl.BlockSpec((L, H * D), lambda i: (0, 0)),
            pl.BlockSpec((L, H * D), lambda i: (0, 0)),
            pl.BlockSpec((L, H * D), lambda i: (0, 0)),
        ],
        out_specs=pl.BlockSpec((H, U, D), lambda i: (0, 0, 0)),
        out_shape=jax.ShapeDtypeStruct((H, U, D), jnp.float32),
    )(idx_all, q2, k2, v2)

    return jnp.transpose(ctx, (1, 0, 2))[None]


def kernel(queries, keys, values, attn_mask):
    return (_run(queries, keys, values), None)


# speedup vs baseline: 1.1161x; 1.0068x over previous
import functools

import jax
import jax.numpy as jnp
import numpy as np
from jax import lax
from jax.experimental import pallas as pl
from jax.experimental.pallas import tpu as pltpu
from jax.experimental.pallas import tpu_sc as plsc

L = 2048
H = 16
D = 64
U = 40
RB = 256
NEG = -1e30

_NT = (((1,), (1,)), ((), ()))


def _rotl(x, d):
    return ((x << np.uint32(d)) | (x >> np.uint32(32 - d))).astype(np.uint32)


def _threefry2x32(k1, k2, x0, x1):
    k1 = np.uint32(k1)
    k2 = np.uint32(k2)
    x0 = x0.astype(np.uint32).copy()
    x1 = x1.astype(np.uint32).copy()
    ks = [k1, k2, k1 ^ k2 ^ np.uint32(0x1BD11BDA)]
    rot = [(13, 15, 26, 6), (17, 29, 16, 24)]
    x0 = x0 + ks[0]
    x1 = x1 + ks[1]
    for rs, a, b, c in [(rot[0], 1, 2, 1), (rot[1], 2, 0, 2), (rot[0], 0, 1, 3),
                        (rot[1], 1, 2, 4), (rot[0], 2, 0, 5)]:
        for r in rs:
            x0 = (x0 + x1).astype(np.uint32)
            x1 = x0 ^ _rotl(x1, r)
        x0 = (x0 + ks[a]).astype(np.uint32)
        x1 = (x1 + ks[b] + np.uint32(c)).astype(np.uint32)
    return x0, x1


def _sample_indices() -> np.ndarray:
    b1, b2 = _threefry2x32(0, 42, np.zeros(2, np.uint32), np.arange(2))
    i = np.arange(L * U, dtype=np.uint64)
    o1, o2 = _threefry2x32(b1[1], b2[1],
                           (i >> np.uint64(32)).astype(np.uint32),
                           (i & np.uint64(0xFFFFFFFF)).astype(np.uint32))
    return ((o1 ^ o2).reshape(L, U) % np.uint32(L)).astype(np.int32)


def _build_counts() -> np.ndarray:
    idx = _sample_indices()
    cnt = np.zeros((L, L), np.float32)
    np.add.at(cnt, (np.arange(L)[:, None], idx), 1.0)
    return cnt


_CNT = _build_counts()


def _m_kernel(q_ref, k_ref, cnt_ref, m_ref):
    cnt = cnt_ref[...]
    pos = cnt > 0.0
    for h in range(H):
        q_h = q_ref[:, h * D:(h + 1) * D]
        k_h = k_ref[:, h * D:(h + 1) * D]
        c = jax.lax.dot_general(q_h, k_h, _NT,
                                preferred_element_type=jnp.float32)
        m_max = jnp.max(jnp.where(pos, c, NEG), axis=1)
        m_sum = jnp.sum(c * cnt, axis=1)
        m_ref[h, 0, :] = m_max - m_sum * (1.0 / L)


UPAD = 48
_NLANE = 16


_GDN = lax.GatherDimensionNumbers(offset_dims=(), collapsed_slice_dims=(0,),
                                  start_index_map=(0,))


def _lane_shuffle(x, idx16):
    return lax.gather(x, idx16[:, None], _GDN, (1,),
                      mode=lax.GatherScatterMode.PROMISE_IN_BOUNDS)


def _sc_topk_kernel(m_hbm, out_hbm, m_v, idx_v):
    wid = lax.axis_index("s") * 2 + lax.axis_index("c")

    @pl.when(wid < H)
    def _():
        pltpu.sync_copy(m_hbm.at[wid], m_v)
        zero16 = jnp.zeros((_NLANE,), jnp.int32)
        for j in range(128 // _NLANE):
            idx_v[pl.ds(j * _NLANE, _NLANE)] = zero16
        lane_iota = lax.iota(jnp.int32, _NLANE)

        def bfly_max(x):
            for sh in (1, 2, 4, 8):
                x = jnp.maximum(x, _lane_shuffle(x, lane_iota ^ sh))
            return x

        def bfly_min(x):
            for sh in (1, 2, 4, 8):
                x = jnp.minimum(x, _lane_shuffle(x, lane_iota ^ sh))
            return x

        def block_max(base):
            bmax = m_v[pl.ds(base, _NLANE)]
            for j in range(1, 8):
                bmax = jnp.maximum(bmax, m_v[pl.ds(base + j * _NLANE, _NLANE)])
            return bfly_max(bmax)

        bm = jnp.full((_NLANE,), NEG, jnp.float32)
        for b in range(L // 128):
            bm = jnp.where(lane_iota == b, block_max(b * 128), bm)

        def pick_one(u, bm):
            cur = bfly_max(bm)
            bmatch = jnp.where(bm == cur, lane_iota, _NLANE)
            b = bfly_min(bmatch)[0]
            base = b * 128
            rmax = jnp.full((_NLANE,), NEG, jnp.float32)
            ridx = jnp.zeros((_NLANE,), jnp.int32)
            for j in range(8):
                v = m_v[pl.ds(base + j * _NLANE, _NLANE)]
                take = v > rmax
                rmax = jnp.where(take, v, rmax)
                ridx = jnp.where(take, base + j * _NLANE + lane_iota, ridx)
            cand = jnp.where(rmax == cur, ridx, L)
            cand = bfly_min(cand)
            idx = cand[0]

            ub = (u // _NLANE) * _NLANE
            ichunk = idx_v[pl.ds(ub, _NLANE)]
            idx_v[pl.ds(ub, _NLANE)] = jnp.where(lane_iota == u - ub,
                                                 cand, ichunk)
            mb = (idx // _NLANE) * _NLANE
            mchunk = m_v[pl.ds(mb, _NLANE)]
            m_v[pl.ds(mb, _NLANE)] = jnp.where(lane_iota == idx - mb,
                                               jnp.float32(NEG), mchunk)
            return jnp.where(lane_iota == b, block_max(base), bm)

        lax.fori_loop(0, U, pick_one, bm)
        pltpu.sync_copy(idx_v, out_hbm.at[wid])


def _sc_topk(m):
    mesh = plsc.VectorSubcoreMesh(core_axis_name="c", subcore_axis_name="s")
    fn = functools.partial(
        pl.kernel,
        out_type=jax.ShapeDtypeStruct((H, 128), jnp.int32),
        mesh=mesh,
        scratch_types=[
            pltpu.VMEM((L,), jnp.float32),
            pltpu.VMEM((128,), jnp.int32),
        ],
    )(_sc_topk_kernel)
    return fn(m)


def _attn_kernel(idx_ref, q_ref, k_ref, v_ref, out_ref):
    idx_all = idx_ref[...]

    iota_l = jax.lax.broadcasted_iota(jnp.int32, (U, L), 1)
    iota_s = jax.lax.broadcasted_iota(jnp.int32, (U, U), 0)
    iota_u = jax.lax.broadcasted_iota(jnp.int32, (U, U), 1)
    for h in range(H):
        q_h = q_ref[:, h * D:(h + 1) * D]
        k_h = k_ref[:, h * D:(h + 1) * D]
        v_h = v_ref[:, h * D:(h + 1) * D]
        idx_row = idx_all[h:h + 1, :]
        idx_col = jnp.transpose(idx_row, (1, 0))

        g = (iota_l == idx_col).astype(jnp.float32)
        q_red = jnp.dot(g, q_h, preferred_element_type=jnp.float32)

        scores = jax.lax.dot_general(
            q_red, k_h, _NT,
            preferred_element_type=jnp.float32) * (1.0 / np.sqrt(D))
        s_max = jnp.max(scores, axis=1, keepdims=True)
        e = jnp.exp(scores - s_max)
        p = e / jnp.sum(e, axis=1, keepdims=True)
        upd = jnp.dot(p, v_h, preferred_element_type=jnp.float32)

        clip_row = jnp.minimum(idx_row, U - 1)
        eq = clip_row == iota_s
        u_star = jnp.max(jnp.where(eq, iota_u, -1), axis=1, keepdims=True)
        w = ((iota_u == u_star) & eq).astype(jnp.float32)
        scat = jnp.dot(w, upd, preferred_element_type=jnp.float32)

        v_mean = jnp.mean(v_h, axis=0, keepdims=True)
        out_ref[h] = jnp.where(u_star < 0,
                               jnp.broadcast_to(v_mean, (U, D)), scat)


@jax.jit
def _run(queries, keys, values):
    q2 = queries.reshape(L, H * D)
    k2 = keys.reshape(L, H * D)
    v2 = values.reshape(L, H * D)
    cnt = jnp.asarray(_CNT)

    m = pl.pallas_call(
        _m_kernel,
        grid=(L // RB,),
        in_specs=[
            pl.BlockSpec((RB, H * D), lambda rb: (rb, 0)),
            pl.BlockSpec((L, H * D), lambda rb: (0, 0)),
            pl.BlockSpec((RB, L), lambda rb: (rb, 0)),
        ],
        out_specs=pl.BlockSpec((H, 1, RB), lambda rb: (0, 0, rb)),
        out_shape=jax.ShapeDtypeStruct((H, 1, L), jnp.float32),
    )(q2, k2, cnt)

    idx_all = _sc_topk(m.reshape(H, L))[:, :U]

    ctx = pl.pallas_call(
        _attn_kernel,
        grid=(1,),
        in_specs=[
            pl.BlockSpec((H, U), lambda i: (0, 0)),
            pl.BlockSpec((L, H * D), lambda i: (0, 0)),
            pl.BlockSpec((L, H * D), lambda i: (0, 0)),
            pl.BlockSpec((L, H * D), lambda i: (0, 0)),
        ],
        out_specs=pl.BlockSpec((H, U, D), lambda i: (0, 0, 0)),
        out_shape=jax.ShapeDtypeStruct((H, U, D), jnp.float32),
    )(idx_all, q2, k2, v2)

    return jnp.transpose(ctx, (1, 0, 2))[None]


def kernel(queries, keys, values, attn_mask):
    return (_run(queries, keys, values), None)


# speedup vs baseline: 1.1485x; 1.0290x over previous
import functools

import jax
import jax.numpy as jnp
import numpy as np
from jax import lax
from jax.experimental import pallas as pl
from jax.experimental.pallas import tpu as pltpu
from jax.experimental.pallas import tpu_sc as plsc

L = 2048
H = 16
D = 64
U = 40
RB = 256
NEG = -1e30

_NT = (((1,), (1,)), ((), ()))


def _rotl(x, d):
    return ((x << np.uint32(d)) | (x >> np.uint32(32 - d))).astype(np.uint32)


def _threefry2x32(k1, k2, x0, x1):
    k1 = np.uint32(k1)
    k2 = np.uint32(k2)
    x0 = x0.astype(np.uint32).copy()
    x1 = x1.astype(np.uint32).copy()
    ks = [k1, k2, k1 ^ k2 ^ np.uint32(0x1BD11BDA)]
    rot = [(13, 15, 26, 6), (17, 29, 16, 24)]
    x0 = x0 + ks[0]
    x1 = x1 + ks[1]
    for rs, a, b, c in [(rot[0], 1, 2, 1), (rot[1], 2, 0, 2), (rot[0], 0, 1, 3),
                        (rot[1], 1, 2, 4), (rot[0], 2, 0, 5)]:
        for r in rs:
            x0 = (x0 + x1).astype(np.uint32)
            x1 = x0 ^ _rotl(x1, r)
        x0 = (x0 + ks[a]).astype(np.uint32)
        x1 = (x1 + ks[b] + np.uint32(c)).astype(np.uint32)
    return x0, x1


def _sample_indices() -> np.ndarray:
    b1, b2 = _threefry2x32(0, 42, np.zeros(2, np.uint32), np.arange(2))
    i = np.arange(L * U, dtype=np.uint64)
    o1, o2 = _threefry2x32(b1[1], b2[1],
                           (i >> np.uint64(32)).astype(np.uint32),
                           (i & np.uint64(0xFFFFFFFF)).astype(np.uint32))
    return ((o1 ^ o2).reshape(L, U) % np.uint32(L)).astype(np.int32)


def _build_counts() -> np.ndarray:
    idx = _sample_indices()
    cnt = np.zeros((L, L), np.float32)
    np.add.at(cnt, (np.arange(L)[:, None], idx), 1.0)
    return cnt


_CNT = _build_counts()


def _m_kernel(q_ref, k_ref, cnt_ref, m_ref):
    cnt = cnt_ref[...]
    pos = cnt > 0.0
    for h in range(H):
        q_h = q_ref[:, h * D:(h + 1) * D]
        k_h = k_ref[:, h * D:(h + 1) * D]
        c = jax.lax.dot_general(q_h, k_h, _NT,
                                preferred_element_type=jnp.float32)
        m_max = jnp.max(jnp.where(pos, c, NEG), axis=1)
        m_sum = jnp.sum(c * cnt, axis=1)
        m_ref[h, 0, :] = m_max - m_sum * (1.0 / L)


UPAD = 48
_NLANE = 16


_GDN = lax.GatherDimensionNumbers(offset_dims=(), collapsed_slice_dims=(0,),
                                  start_index_map=(0,))


def _lane_shuffle(x, idx16):
    return lax.gather(x, idx16[:, None], _GDN, (1,),
                      mode=lax.GatherScatterMode.PROMISE_IN_BOUNDS)


def _sc_topk_kernel(m_hbm, out_hbm, m_v, idx_v):
    wid = lax.axis_index("s") * 2 + lax.axis_index("c")

    @pl.when(wid < H)
    def _():
        pltpu.sync_copy(m_hbm.at[wid], m_v)
        zero16 = jnp.zeros((_NLANE,), jnp.int32)
        for j in range(128 // _NLANE):
            idx_v[pl.ds(j * _NLANE, _NLANE)] = zero16
        lane_iota = lax.iota(jnp.int32, _NLANE)

        def bfly_max(x):
            for sh in (1, 2, 4, 8):
                x = jnp.maximum(x, _lane_shuffle(x, lane_iota ^ sh))
            return x

        def bfly_min(x):
            for sh in (1, 2, 4, 8):
                x = jnp.minimum(x, _lane_shuffle(x, lane_iota ^ sh))
            return x

        def block_max(base):
            bmax = m_v[pl.ds(base, _NLANE)]
            for j in range(1, 8):
                bmax = jnp.maximum(bmax, m_v[pl.ds(base + j * _NLANE, _NLANE)])
            return bfly_max(bmax)

        bm = jnp.full((_NLANE,), NEG, jnp.float32)
        for b in range(L // 128):
            bm = jnp.where(lane_iota == b, block_max(b * 128), bm)

        def pick_one(u, bm):
            cur = bfly_max(bm)
            bmatch = jnp.where(bm == cur, lane_iota, _NLANE)
            b = bfly_min(bmatch)[0]
            base = b * 128
            rmax = jnp.full((_NLANE,), NEG, jnp.float32)
            ridx = jnp.zeros((_NLANE,), jnp.int32)
            for j in range(8):
                v = m_v[pl.ds(base + j * _NLANE, _NLANE)]
                take = v > rmax
                rmax = jnp.where(take, v, rmax)
                ridx = jnp.where(take, base + j * _NLANE + lane_iota, ridx)
            cand = jnp.where(rmax == cur, ridx, L)
            cand = bfly_min(cand)
            idx = cand[0]

            ub = (u // _NLANE) * _NLANE
            ichunk = idx_v[pl.ds(ub, _NLANE)]
            idx_v[pl.ds(ub, _NLANE)] = jnp.where(lane_iota == u - ub,
                                                 cand, ichunk)
            mb = (idx // _NLANE) * _NLANE
            mchunk = m_v[pl.ds(mb, _NLANE)]
            m_v[pl.ds(mb, _NLANE)] = jnp.where(lane_iota == idx - mb,
                                               jnp.float32(NEG), mchunk)
            return jnp.where(lane_iota == b, block_max(base), bm)

        lax.fori_loop(0, U, pick_one, bm)
        pltpu.sync_copy(idx_v, out_hbm.at[wid])


def _sc_topk(m):
    mesh = plsc.VectorSubcoreMesh(core_axis_name="c", subcore_axis_name="s")
    fn = functools.partial(
        pl.kernel,
        out_type=jax.ShapeDtypeStruct((H, 128), jnp.int32),
        mesh=mesh,
        scratch_types=[
            pltpu.VMEM((L,), jnp.float32),
            pltpu.VMEM((128,), jnp.int32),
        ],
    )(_sc_topk_kernel)
    return fn(m)


def _attn_kernel(idx_ref, q_ref, k_ref, v_ref, out_ref):
    hp = pl.program_id(0)

    iota_l = jax.lax.broadcasted_iota(jnp.int32, (U, L), 1)
    iota_s = jax.lax.broadcasted_iota(jnp.int32, (U, U), 0)
    iota_u = jax.lax.broadcasted_iota(jnp.int32, (U, U), 1)
    for h in range(2):
        q_h = q_ref[:, h * D:(h + 1) * D]
        k_h = k_ref[:, h * D:(h + 1) * D]
        v_h = v_ref[:, h * D:(h + 1) * D]
        idx_row = idx_ref[pl.ds(2 * hp + h, 1), :]
        idx_col = jnp.transpose(idx_row, (1, 0))

        g = (iota_l == idx_col).astype(jnp.float32)
        q_red = jnp.dot(g, q_h, preferred_element_type=jnp.float32)

        scores = jax.lax.dot_general(
            q_red, k_h, _NT,
            preferred_element_type=jnp.float32) * (1.0 / np.sqrt(D))
        s_max = jnp.max(scores, axis=1, keepdims=True)
        e = jnp.exp(scores - s_max)
        p = e / jnp.sum(e, axis=1, keepdims=True)
        upd = jnp.dot(p, v_h, preferred_element_type=jnp.float32)

        clip_row = jnp.minimum(idx_row, U - 1)
        eq = clip_row == iota_s
        u_star = jnp.max(jnp.where(eq, iota_u, -1), axis=1, keepdims=True)
        w = ((iota_u == u_star) & eq).astype(jnp.float32)
        scat = jnp.dot(w, upd, preferred_element_type=jnp.float32)

        v_mean = jnp.mean(v_h, axis=0, keepdims=True)
        out_ref[h] = jnp.where(u_star < 0,
                               jnp.broadcast_to(v_mean, (U, D)), scat)


_HP = 2 * D


@jax.jit
def _run(queries, keys, values):
    q2 = queries.reshape(L, H * D)
    k2 = keys.reshape(L, H * D)
    v2 = values.reshape(L, H * D)
    cnt = jnp.asarray(_CNT)

    m = pl.pallas_call(
        _m_kernel,
        grid=(L // RB,),
        in_specs=[
            pl.BlockSpec((RB, H * D), lambda rb: (rb, 0)),
            pl.BlockSpec((L, H * D), lambda rb: (0, 0)),
            pl.BlockSpec((RB, L), lambda rb: (rb, 0)),
        ],
        out_specs=pl.BlockSpec((H, 1, RB), lambda rb: (0, 0, rb)),
        out_shape=jax.ShapeDtypeStruct((H, 1, L), jnp.float32),
    )(q2, k2, cnt)

    idx_all = _sc_topk(m.reshape(H, L))[:, :U]

    ctx = pl.pallas_call(
        _attn_kernel,
        grid=(H // 2,),
        in_specs=[
            pl.BlockSpec((H, U), lambda hp: (0, 0)),
            pl.BlockSpec((L, _HP), lambda hp: (0, hp)),
            pl.BlockSpec((L, _HP), lambda hp: (0, hp)),
            pl.BlockSpec((L, _HP), lambda hp: (0, hp)),
        ],
        out_specs=pl.BlockSpec((2, U, D), lambda hp: (hp, 0, 0)),
        out_shape=jax.ShapeDtypeStruct((H, U, D), jnp.float32),
    )(idx_all, q2, k2, v2)

    return jnp.transpose(ctx, (1, 0, 2))[None]


def kernel(queries, keys, values, attn_mask):
    return (_run(queries, keys, values), None)
